# Initial kernel scaffold; baseline (speedup 1.0000x reference)
import jax
import jax.numpy as jnp
from jax.experimental import pallas as pl


def kernel(x, edge_index, W1, b1, W2, b2):
    raise NotImplementedError("write your pallas kernel here")


# speedup vs baseline: 4.3279x; 4.3279x over previous
import functools

import jax
import jax.numpy as jnp
from jax import lax
from jax.experimental import pallas as pl
from jax.experimental.pallas import tpu as pltpu
from jax.experimental.pallas import tpu_sc as plsc

N = 10000
E = 320000
D = 128
H = 256

NUM_TILES = 32
CHUNK = 128
CHUNKS_PER_TILE = 79
EDGES_PER_TILE = CHUNK * CHUNKS_PER_TILE
EPAD = NUM_TILES * EDGES_PER_TILE
NPAD = 10112
ZROWS = NPAD // 16


def _sc_body(x_hbm, send_hbm, recv_hbm, zeros_hbm, out_hbm,
             agg_sh, send_v, recv_v, rows_v, sem):
    c = lax.axis_index("c")
    s = lax.axis_index("s")
    wid = c * 16 + s

    pltpu.sync_copy(zeros_hbm, agg_sh.at[pl.ds(s * ZROWS, ZROWS)])
    plsc.subcore_barrier()

    ebase = wid * EDGES_PER_TILE

    def step(j, carry):
        base = pl.multiple_of(ebase + j * CHUNK, 8)
        pltpu.sync_copy(send_hbm.at[pl.ds(base, CHUNK)], send_v)
        pltpu.sync_copy(recv_hbm.at[pl.ds(base, CHUNK)], recv_v)
        pltpu.async_copy(x_hbm.at[send_v], rows_v, sem).wait()
        pltpu.sync_copy(rows_v, agg_sh.at[recv_v], add=True)
        return carry

    lax.fori_loop(0, CHUNKS_PER_TILE, step, 0)
    plsc.subcore_barrier()

    r0 = s * ZROWS
    pltpu.sync_copy(agg_sh.at[pl.ds(r0, ZROWS)],
                    out_hbm.at[c, pl.ds(r0, ZROWS)])


_sc_aggregate = functools.partial(
    pl.kernel,
    out_type=jax.ShapeDtypeStruct((2, NPAD, D), jnp.float32),
    mesh=plsc.VectorSubcoreMesh(core_axis_name="c", subcore_axis_name="s"),
    scratch_types=[
        pltpu.VMEM_SHARED((NPAD, D), jnp.float32),
        pltpu.VMEM((CHUNK,), jnp.int32),
        pltpu.VMEM((CHUNK,), jnp.int32),
        pltpu.VMEM((CHUNK, D), jnp.float32),
        pltpu.SemaphoreType.DMA,
    ],
)(_sc_body)


def _mlp_body(x_ref, p_ref, w1_ref, b1_ref, w2_ref, b2_ref, o_ref):
    agg = p_ref[0] + p_ref[1]
    h = (
        jnp.dot(x_ref[...], w1_ref[:D, :], preferred_element_type=jnp.float32)
        + jnp.dot(agg, w1_ref[D:, :], preferred_element_type=jnp.float32)
        + b1_ref[...]
    )
    h = jnp.maximum(h, 0.0)
    o_ref[...] = (
        jnp.dot(h, w2_ref[...], preferred_element_type=jnp.float32)
        + b2_ref[...]
    )


def _mlp(x, partials, W1, b1, W2, b2):
    blk = 2000
    grid = (N // blk,)
    return pl.pallas_call(
        _mlp_body,
        grid=grid,
        in_specs=[
            pl.BlockSpec((blk, D), lambda i: (i, 0)),
            pl.BlockSpec((2, blk, D), lambda i: (0, i, 0)),
            pl.BlockSpec((2 * D, H), lambda i: (0, 0)),
            pl.BlockSpec((1, H), lambda i: (0, 0)),
            pl.BlockSpec((H, D), lambda i: (0, 0)),
            pl.BlockSpec((1, D), lambda i: (0, 0)),
        ],
        out_specs=pl.BlockSpec((blk, D), lambda i: (i, 0)),
        out_shape=jax.ShapeDtypeStruct((N, D), jnp.float32),
    )(x, partials, W1, b1, W2, b2)


def kernel(x, edge_index, W1, b1, W2, b2):
    senders = edge_index[0]
    receivers = edge_index[1]
    pad = EPAD - E
    senders_p = jnp.concatenate(
        [senders, jnp.zeros((pad,), dtype=jnp.int32)])
    receivers_p = jnp.concatenate(
        [receivers, jnp.full((pad,), N, dtype=jnp.int32)])
    zeros = jnp.zeros((ZROWS, D), dtype=jnp.float32)
    partials = _sc_aggregate(x, senders_p, receivers_p, zeros)[:, :N]
    return _mlp(x, partials, W1, b1.reshape(1, H), W2, b2.reshape(1, D))
